# baseline (speedup 1.0000x reference)
import jax
import jax.numpy as jnp
from jax.experimental import pallas as pl
from jax.experimental.pallas import tpu as pltpu

MODEL_DIM = 768
FFN_DIM = 768
NUM_EXPERTS = 64
CAPACITY = 64
TOKENS = 2 * 2048
TB = 1024
NBLK = TOKENS // TB


def _moe_body(xfb_ref, wr_ref, wg_ref, bg_ref, wv_ref, bv_ref, wo_ref,
              bo_ref, logits_ref, probs_ref, ptb_ref, ei_ref, out_ref,
              carry_ref, acc_ref, xfc_ref, xi_ref, eiv_ref, ptv_ref,
              ism_ref, psm_ref, sem_i, sem_p):
    g = pl.program_id(0)
    E, C = NUM_EXPERTS, CAPACITY

    @pl.when(g == 0)
    def _init0():
        carry_ref[...] = jnp.zeros_like(carry_ref)
        acc_ref[...] = jnp.zeros_like(acc_ref)
        out_ref[...] = jnp.zeros_like(out_ref)

    @pl.when(g < NBLK)
    def _dispatch():
        xb = xfb_ref[...]
        xfc_ref[pl.ds(g * TB, TB), :] = xb
        logits = jnp.dot(xb, wr_ref[...], preferred_element_type=jnp.float32)
        logits_ref[...] = logits
        m = jnp.max(logits, axis=1, keepdims=True)
        ex = jnp.exp(logits - m)
        probs = ex / jnp.sum(ex, axis=1, keepdims=True)
        probs_ref[...] = probs

        lane = jax.lax.broadcasted_iota(jnp.int32, (TB, E), 1)
        top_i = jnp.min(jnp.where(logits == m, lane, E), axis=1)
        top_p = jnp.max(probs, axis=1)
        oh_e = (lane == top_i[:, None]).astype(jnp.float32)

        cs = oh_e
        k = 1
        while k < TB:
            cs = cs + jnp.concatenate(
                [jnp.zeros((k, E), jnp.float32), cs[:-k, :]], axis=0)
            k *= 2
        pos_mat = cs - oh_e + carry_ref[0:1, :]
        carry_ref[0:1, :] = carry_ref[0:1, :] + cs[TB - 1:TB, :]
        pos_t = jnp.sum(pos_mat * oh_e, axis=1)

        cap_lane = jax.lax.broadcasted_iota(jnp.int32, (TB, C), 1)
        oh_c = (cap_lane == pos_t.astype(jnp.int32)[:, None]).astype(jnp.float32)
        tok_id = (jax.lax.broadcasted_iota(jnp.int32, (TB, 1), 0)
                  + g * TB).astype(jnp.float32)
        rhs = jnp.concatenate(
            [oh_c, oh_c * tok_id, oh_c * top_p[:, None]], axis=1)
        dn = (((0,), (0,)), ((), ()))
        acc_ref[...] += jax.lax.dot_general(
            oh_e, rhs, dn, precision=jax.lax.Precision.HIGHEST,
            preferred_element_type=jnp.float32)

        @pl.when(g == NBLK - 1)
        def _finalize():
            acc = acc_ref[...]
            filled = acc[:, :C] > 0.5
            ei = jnp.where(filled, acc[:, C:2 * C].astype(jnp.int32), -1)
            ptb_ref[...] = acc[:, 2 * C:3 * C]
            ei_ref[...] = ei
            eiv_ref[...] = ei
            ptv_ref[...] = acc[:, 2 * C:3 * C]
            pltpu.make_async_copy(eiv_ref, ism_ref, sem_i).start()
            pltpu.make_async_copy(ptv_ref, psm_ref, sem_p).start()

    @pl.when(g >= NBLK)
    def _expert():
        e = g - NBLK

        @pl.when(g == NBLK)
        def _arrive():
            pltpu.make_async_copy(eiv_ref, ism_ref, sem_i).wait()
            pltpu.make_async_copy(ptv_ref, psm_ref, sem_p).wait()

        for i in range(C):
            t = ism_ref[e, i]
            ts = jnp.maximum(t, 0)
            xi_ref[pl.ds(i, 1), :] = xfc_ref[pl.ds(ts, 1), :]

        xi = xi_ref[...]
        gate = (jnp.dot(xi, wg_ref[0], preferred_element_type=jnp.float32)
                + bg_ref[pl.ds(e, 1), :])
        val = (jnp.dot(xi, wv_ref[0], preferred_element_type=jnp.float32)
               + bv_ref[pl.ds(e, 1), :])
        h = val * (gate * jax.nn.sigmoid(gate))
        eo = (jnp.dot(h, wo_ref[0], preferred_element_type=jnp.float32)
              + bo_ref[pl.ds(e, 1), :])

        for i in range(C):
            t = ism_ref[e, i]
            row = eo[i:i + 1, :] * psm_ref[e, i]

            @pl.when(t >= 0)
            def _store(row=row, t=t):
                out_ref[pl.ds(t, 1), :] = row


@jax.jit
def kernel(x, W_router, Wg, bg, Wv, bv, Wo, bo):
    B, S, D = x.shape
    T = B * S
    E, C, F = NUM_EXPERTS, CAPACITY, FFN_DIM
    xf = x.reshape(T, D)

    blk16 = NBLK - 1
    logits, probs, ptb, ei, out = pl.pallas_call(
        _moe_body,
        grid=(NBLK + E,),
        in_specs=[
            pl.BlockSpec((TB, D), lambda g: (jnp.minimum(g, blk16), 0)),
            pl.BlockSpec((D, E), lambda g: (0, 0)),
            pl.BlockSpec((1, D, F), lambda g: (jnp.maximum(g - NBLK, 0), 0, 0)),
            pl.BlockSpec((E, F), lambda g: (0, 0)),
            pl.BlockSpec((1, D, F), lambda g: (jnp.maximum(g - NBLK, 0), 0, 0)),
            pl.BlockSpec((E, F), lambda g: (0, 0)),
            pl.BlockSpec((1, F, D), lambda g: (jnp.maximum(g - NBLK, 0), 0, 0)),
            pl.BlockSpec((E, D), lambda g: (0, 0)),
        ],
        out_specs=[
            pl.BlockSpec((TB, E), lambda g: (jnp.minimum(g, blk16), 0)),
            pl.BlockSpec((TB, E), lambda g: (jnp.minimum(g, blk16), 0)),
            pl.BlockSpec((E, C), lambda g: (0, 0)),
            pl.BlockSpec((E, C), lambda g: (0, 0)),
            pl.BlockSpec((T, D), lambda g: (0, 0)),
        ],
        out_shape=[
            jax.ShapeDtypeStruct((T, E), jnp.float32),
            jax.ShapeDtypeStruct((T, E), jnp.float32),
            jax.ShapeDtypeStruct((E, C), jnp.float32),
            jax.ShapeDtypeStruct((E, C), jnp.int32),
            jax.ShapeDtypeStruct((T, D), jnp.float32),
        ],
        scratch_shapes=[
            pltpu.VMEM((8, E), jnp.float32),
            pltpu.VMEM((E, 3 * C), jnp.float32),
            pltpu.VMEM((T, D), jnp.float32),
            pltpu.VMEM((C, D), jnp.float32),
            pltpu.VMEM((E, C), jnp.int32),
            pltpu.VMEM((E, C), jnp.float32),
            pltpu.SMEM((E, C), jnp.int32),
            pltpu.SMEM((E, C), jnp.float32),
            pltpu.SemaphoreType.DMA,
            pltpu.SemaphoreType.DMA,
        ],
    )(xf, W_router, Wg, bg, Wv, bv, Wo, bo)

    return (out.reshape(B, S, D), logits, probs, ptb, ei)


# speedup vs baseline: 1.0008x; 1.0008x over previous
import jax
import jax.numpy as jnp
from jax.experimental import pallas as pl
from jax.experimental.pallas import tpu as pltpu

MODEL_DIM = 768
FFN_DIM = 768
NUM_EXPERTS = 64
CAPACITY = 64
TOKENS = 2 * 2048
TB = 512
NBLK = TOKENS // TB


def _moe_body(xfb_ref, wr_ref, wg_ref, bg_ref, wv_ref, bv_ref, wo_ref,
              bo_ref, logits_ref, probs_ref, ptb_ref, ei_ref, out_ref,
              carry_ref, acc_ref, xfc_ref, xi_ref, eiv_ref, ptv_ref,
              ism_ref, psm_ref, sem_i, sem_p):
    g = pl.program_id(0)
    E, C = NUM_EXPERTS, CAPACITY

    @pl.when(g == 0)
    def _init0():
        carry_ref[...] = jnp.zeros_like(carry_ref)
        acc_ref[...] = jnp.zeros_like(acc_ref)
        out_ref[...] = jnp.zeros_like(out_ref)

    @pl.when(g < NBLK)
    def _dispatch():
        xb = xfb_ref[...]
        xfc_ref[pl.ds(g * TB, TB), :] = xb
        logits = jnp.dot(xb, wr_ref[...], preferred_element_type=jnp.float32)
        logits_ref[...] = logits
        m = jnp.max(logits, axis=1, keepdims=True)
        ex = jnp.exp(logits - m)
        probs = ex / jnp.sum(ex, axis=1, keepdims=True)
        probs_ref[...] = probs

        lane = jax.lax.broadcasted_iota(jnp.int32, (TB, E), 1)
        top_i = jnp.min(jnp.where(logits == m, lane, E), axis=1)
        top_p = jnp.max(probs, axis=1)
        oh_e = (lane == top_i[:, None]).astype(jnp.float32)

        cs = oh_e
        k = 1
        while k < TB:
            cs = cs + jnp.concatenate(
                [jnp.zeros((k, E), jnp.float32), cs[:-k, :]], axis=0)
            k *= 2
        pos_mat = cs - oh_e + carry_ref[0:1, :]
        carry_ref[0:1, :] = carry_ref[0:1, :] + cs[TB - 1:TB, :]
        pos_t = jnp.sum(pos_mat * oh_e, axis=1)

        cap_lane = jax.lax.broadcasted_iota(jnp.int32, (TB, C), 1)
        oh_c = (cap_lane == pos_t.astype(jnp.int32)[:, None]).astype(jnp.float32)
        tok_id = (jax.lax.broadcasted_iota(jnp.int32, (TB, 1), 0)
                  + g * TB).astype(jnp.float32)
        rhs = jnp.concatenate(
            [oh_c, oh_c * tok_id, oh_c * top_p[:, None]], axis=1)
        dn = (((0,), (0,)), ((), ()))
        acc_ref[...] += jax.lax.dot_general(
            oh_e, rhs, dn, precision=jax.lax.Precision.HIGHEST,
            preferred_element_type=jnp.float32)

        @pl.when(g == NBLK - 1)
        def _finalize():
            acc = acc_ref[...]
            filled = acc[:, :C] > 0.5
            ei = jnp.where(filled, acc[:, C:2 * C].astype(jnp.int32), -1)
            ptb_ref[...] = acc[:, 2 * C:3 * C]
            ei_ref[...] = ei
            eiv_ref[...] = ei
            ptv_ref[...] = acc[:, 2 * C:3 * C]
            pltpu.make_async_copy(eiv_ref, ism_ref, sem_i).start()
            pltpu.make_async_copy(ptv_ref, psm_ref, sem_p).start()

    @pl.when(g >= NBLK)
    def _expert():
        e = g - NBLK

        @pl.when(g == NBLK)
        def _arrive():
            pltpu.make_async_copy(eiv_ref, ism_ref, sem_i).wait()
            pltpu.make_async_copy(ptv_ref, psm_ref, sem_p).wait()

        for i in range(C):
            t = ism_ref[e, i]
            ts = jnp.maximum(t, 0)
            xi_ref[pl.ds(i, 1), :] = xfc_ref[pl.ds(ts, 1), :]

        xi = xi_ref[...]
        gate = (jnp.dot(xi, wg_ref[0], preferred_element_type=jnp.float32)
                + bg_ref[pl.ds(e, 1), :])
        val = (jnp.dot(xi, wv_ref[0], preferred_element_type=jnp.float32)
               + bv_ref[pl.ds(e, 1), :])
        h = val * (gate * jax.nn.sigmoid(gate))
        eo = (jnp.dot(h, wo_ref[0], preferred_element_type=jnp.float32)
              + bo_ref[pl.ds(e, 1), :])

        for i in range(C):
            t = ism_ref[e, i]
            row = eo[i:i + 1, :] * psm_ref[e, i]

            @pl.when(t >= 0)
            def _store(row=row, t=t):
                out_ref[pl.ds(t, 1), :] = row


@jax.jit
def kernel(x, W_router, Wg, bg, Wv, bv, Wo, bo):
    B, S, D = x.shape
    T = B * S
    E, C, F = NUM_EXPERTS, CAPACITY, FFN_DIM
    xf = x.reshape(T, D)

    blk16 = NBLK - 1
    logits, probs, ptb, ei, out = pl.pallas_call(
        _moe_body,
        grid=(NBLK + E,),
        in_specs=[
            pl.BlockSpec((TB, D), lambda g: (jnp.minimum(g, blk16), 0)),
            pl.BlockSpec((D, E), lambda g: (0, 0)),
            pl.BlockSpec((1, D, F), lambda g: (jnp.maximum(g - NBLK, 0), 0, 0)),
            pl.BlockSpec((E, F), lambda g: (0, 0)),
            pl.BlockSpec((1, D, F), lambda g: (jnp.maximum(g - NBLK, 0), 0, 0)),
            pl.BlockSpec((E, F), lambda g: (0, 0)),
            pl.BlockSpec((1, F, D), lambda g: (jnp.maximum(g - NBLK, 0), 0, 0)),
            pl.BlockSpec((E, D), lambda g: (0, 0)),
        ],
        out_specs=[
            pl.BlockSpec((TB, E), lambda g: (jnp.minimum(g, blk16), 0)),
            pl.BlockSpec((TB, E), lambda g: (jnp.minimum(g, blk16), 0)),
            pl.BlockSpec((E, C), lambda g: (0, 0)),
            pl.BlockSpec((E, C), lambda g: (0, 0)),
            pl.BlockSpec((T, D), lambda g: (0, 0)),
        ],
        out_shape=[
            jax.ShapeDtypeStruct((T, E), jnp.float32),
            jax.ShapeDtypeStruct((T, E), jnp.float32),
            jax.ShapeDtypeStruct((E, C), jnp.float32),
            jax.ShapeDtypeStruct((E, C), jnp.int32),
            jax.ShapeDtypeStruct((T, D), jnp.float32),
        ],
        scratch_shapes=[
            pltpu.VMEM((8, E), jnp.float32),
            pltpu.VMEM((E, 3 * C), jnp.float32),
            pltpu.VMEM((T, D), jnp.float32),
            pltpu.VMEM((C, D), jnp.float32),
            pltpu.VMEM((E, C), jnp.int32),
            pltpu.VMEM((E, C), jnp.float32),
            pltpu.SMEM((E, C), jnp.int32),
            pltpu.SMEM((E, C), jnp.float32),
            pltpu.SemaphoreType.DMA,
            pltpu.SemaphoreType.DMA,
        ],
    )(xf, W_router, Wg, bg, Wv, bv, Wo, bo)

    return (out.reshape(B, S, D), logits, probs, ptb, ei)
